# baseline (speedup 1.0000x reference)
import functools

import jax
import jax.numpy as jnp
from jax import lax
from jax.experimental import pallas as pl
from jax.experimental.pallas import tpu as pltpu
from jax.experimental.pallas import tpu_sc as plsc

B = 16384
D = 1024
NC = 2
NS = 16
NW = NC * NS
BPW = B // NW
C = 32
NCHUNK = BPW // C
NBUF = 3
PRIME = 2
WPR = 4096 // BPW
LANES = 16
SCALE = 32.0

_mesh = plsc.VectorSubcoreMesh(core_axis_name="c", subcore_axis_name="s")


@functools.partial(
    pl.kernel,
    mesh=_mesh,
    out_type=jax.ShapeDtypeStruct((4, 4096, D), jnp.float32),
    scratch_types=[
        pltpu.VMEM((BPW,), jnp.int32),
    ]
    + [pltpu.VMEM((C, D), jnp.float32) for _ in range(NBUF)]
    + [pltpu.SemaphoreType.DMA for _ in range(2 * NBUF)],
)
def _embed(idx_hbm, table_hbm, out_hbm, idx_v, *rest):
    bufs = rest[:NBUF]
    gsems = rest[NBUF : 2 * NBUF]
    ssems = rest[2 * NBUF :]
    wid = lax.axis_index("s") * NC + lax.axis_index("c")
    row = wid // WPR
    off = (wid % WPR) * BPW
    pltpu.sync_copy(idx_hbm.at[row, pl.ds(off, BPW)], idx_v)

    def gather(c):
        b = c % NBUF
        return pltpu.async_copy(
            table_hbm.at[idx_v.at[pl.ds(c * C, C)]], bufs[b], gsems[b]
        )

    def scatter(c):
        b = c % NBUF
        return pltpu.async_copy(
            bufs[b], out_hbm.at[row, pl.ds(off + c * C, C)], ssems[b]
        )

    def scale(buf):
        def scale_row(j, carry):
            for k in range(D // LANES):
                sl = pl.ds(k * LANES, LANES)
                buf[j, sl] = buf[j, sl] * SCALE
            return carry

        lax.fori_loop(0, C, scale_row, 0)

    gh = {}
    sh = {}
    for c in range(PRIME):
        gh[c] = gather(c)
    for c in range(NCHUNK):
        b = c % NBUF
        g = c + PRIME
        if g < NCHUNK:
            if g >= NBUF:
                sh[g - NBUF].wait()
            gh[g] = gather(g)
        gh[c].wait()
        scale(bufs[b])
        sh[c] = scatter(c)
    for c in range(NCHUNK - NBUF, NCHUNK):
        sh[c].wait()


def kernel(token_ids, table):
    return _embed(token_ids, table)


# speedup vs baseline: 1.1075x; 1.1075x over previous
import functools

import jax
import jax.numpy as jnp
from jax import lax
from jax.experimental import pallas as pl
from jax.experimental.pallas import tpu as pltpu
from jax.experimental.pallas import tpu_sc as plsc

B = 16384
D = 1024
NC = 2
NS = 16
NW = NC * NS
BPW = B // NW
C = 16
NCHUNK = BPW // C
NBUF = 4
PRIME = 2
WPR = 4096 // BPW
LANES = 16
SCALE = 32.0

_mesh = plsc.VectorSubcoreMesh(core_axis_name="c", subcore_axis_name="s")


@functools.partial(
    pl.kernel,
    mesh=_mesh,
    out_type=jax.ShapeDtypeStruct((4, 4096, D), jnp.float32),
    scratch_types=[
        pltpu.VMEM((BPW,), jnp.int32),
    ]
    + [pltpu.VMEM((C, D), jnp.float32) for _ in range(NBUF)]
    + [pltpu.SemaphoreType.DMA for _ in range(2 * NBUF)],
)
def _embed(idx_hbm, table_hbm, out_hbm, idx_v, *rest):
    bufs = rest[:NBUF]
    gsems = rest[NBUF : 2 * NBUF]
    ssems = rest[2 * NBUF :]
    wid = lax.axis_index("s") * NC + lax.axis_index("c")
    row = wid // WPR
    off = (wid % WPR) * BPW
    pltpu.sync_copy(idx_hbm.at[row, pl.ds(off, BPW)], idx_v)

    def gather(c):
        b = c % NBUF
        return pltpu.async_copy(
            table_hbm.at[idx_v.at[pl.ds(c * C, C)]], bufs[b], gsems[b]
        )

    def scatter(c):
        b = c % NBUF
        return pltpu.async_copy(
            bufs[b], out_hbm.at[row, pl.ds(off + c * C, C)], ssems[b]
        )

    def scale(buf):
        def scale_row(j, carry):
            for k in range(D // LANES):
                sl = pl.ds(k * LANES, LANES)
                buf[j, sl] = buf[j, sl] * SCALE
            return carry

        lax.fori_loop(0, C, scale_row, 0)

    gh = {}
    sh = {}
    for c in range(PRIME):
        gh[c] = gather(c)
    for c in range(NCHUNK):
        b = c % NBUF
        g = c + PRIME
        if g < NCHUNK:
            if g >= NBUF:
                sh[g - NBUF].wait()
            gh[g] = gather(g)
        gh[c].wait()
        scale(bufs[b])
        sh[c] = scatter(c)
    for c in range(NCHUNK - NBUF, NCHUNK):
        sh[c].wait()


def kernel(token_ids, table):
    return _embed(token_ids, table)


# speedup vs baseline: 1.1118x; 1.0038x over previous
import functools

import jax
import jax.numpy as jnp
from jax import lax
from jax.experimental import pallas as pl
from jax.experimental.pallas import tpu as pltpu
from jax.experimental.pallas import tpu_sc as plsc

B = 16384
D = 1024
NC = 2
NS = 16
NW = NC * NS
BPW = B // NW
C = 16
NCHUNK = BPW // C
NBUF = 6
PRIME = 3
WPR = 4096 // BPW
LANES = 16
SCALE = 32.0

_mesh = plsc.VectorSubcoreMesh(core_axis_name="c", subcore_axis_name="s")


@functools.partial(
    pl.kernel,
    mesh=_mesh,
    out_type=jax.ShapeDtypeStruct((4, 4096, D), jnp.float32),
    scratch_types=[
        pltpu.VMEM((BPW,), jnp.int32),
    ]
    + [pltpu.VMEM((C, D), jnp.float32) for _ in range(NBUF)]
    + [pltpu.SemaphoreType.DMA for _ in range(2 * NBUF)],
)
def _embed(idx_hbm, table_hbm, out_hbm, idx_v, *rest):
    bufs = rest[:NBUF]
    gsems = rest[NBUF : 2 * NBUF]
    ssems = rest[2 * NBUF :]
    wid = lax.axis_index("s") * NC + lax.axis_index("c")
    row = wid // WPR
    off = (wid % WPR) * BPW
    pltpu.sync_copy(idx_hbm.at[row, pl.ds(off, BPW)], idx_v)

    def gather(c):
        b = c % NBUF
        return pltpu.async_copy(
            table_hbm.at[idx_v.at[pl.ds(c * C, C)]], bufs[b], gsems[b]
        )

    def scatter(c):
        b = c % NBUF
        return pltpu.async_copy(
            bufs[b], out_hbm.at[row, pl.ds(off + c * C, C)], ssems[b]
        )

    def scale(buf):
        def scale_row(j, carry):
            for k in range(D // LANES):
                sl = pl.ds(k * LANES, LANES)
                buf[j, sl] = buf[j, sl] * SCALE
            return carry

        lax.fori_loop(0, C, scale_row, 0)

    gh = {}
    sh = {}
    for c in range(PRIME):
        gh[c] = gather(c)
    for c in range(NCHUNK):
        b = c % NBUF
        g = c + PRIME
        if g < NCHUNK:
            if g >= NBUF:
                sh[g - NBUF].wait()
            gh[g] = gather(g)
        gh[c].wait()
        scale(bufs[b])
        sh[c] = scatter(c)
    for c in range(NCHUNK - NBUF, NCHUNK):
        sh[c].wait()


def kernel(token_ids, table):
    return _embed(token_ids, table)
